# baseline (speedup 1.0000x reference)
import jax
import jax.numpy as jnp
from jax.experimental import pallas as pl

C = 64
G = 8
B = 4096
P = 49
PCHUNK = 1
NSTEPS = P // PCHUNK
MOM = 0.1
INV_COUNT = 1.0 / (G * B)


def _body(x_ref, gm_ref, o_ref):
    blk = x_ref[...]
    part = jnp.sum(blk.reshape(PCHUNK, G, G, B), axis=(2, 3))
    o_ref[0] = (1.0 - MOM) * gm_ref[0] + (MOM * INV_COUNT) * part


def kernel(query_attention_maps, group_memory):
    x = jnp.transpose(query_attention_maps, (0, 3, 4, 1, 2)).reshape(P, C, B)
    gm_t = group_memory.reshape(G, P).T.reshape(NSTEPS, PCHUNK, G)
    res = pl.pallas_call(
        _body,
        grid=(NSTEPS,),
        in_specs=[
            pl.BlockSpec((PCHUNK, C, B), lambda j: (j, 0, 0)),
            pl.BlockSpec((1, PCHUNK, G), lambda j: (j, 0, 0)),
        ],
        out_specs=pl.BlockSpec((1, PCHUNK, G), lambda j: (j, 0, 0)),
        out_shape=jax.ShapeDtypeStruct((NSTEPS, PCHUNK, G), jnp.float32),
    )(x, gm_t)
    return res.reshape(P, G).T.reshape(G, 1, 7, 7)


# speedup vs baseline: 2.0442x; 2.0442x over previous
import jax
import jax.numpy as jnp
from jax.experimental import pallas as pl

C = 64
G = 8
B = 4096
P = 49
PCHUNK = 7
NSTEPS = P // PCHUNK
NSPLIT = 4
CS = C // NSPLIT
GS = G // NSPLIT
MOM = 0.1
INV_COUNT = 1.0 / (G * B)


def _body(x0_ref, x1_ref, x2_ref, x3_ref, gm_ref, o_ref):
    parts = []
    for r in (x0_ref, x1_ref, x2_ref, x3_ref):
        blk = r[...]
        parts.append(jnp.sum(blk.reshape(PCHUNK, GS, G, B), axis=(2, 3)))
    part = jnp.concatenate(parts, axis=1)
    o_ref[0] = (1.0 - MOM) * gm_ref[0] + (MOM * INV_COUNT) * part


def kernel(query_attention_maps, group_memory):
    x = jnp.transpose(query_attention_maps, (0, 3, 4, 1, 2)).reshape(P, C, B)
    gm_t = group_memory.reshape(G, P).T.reshape(NSTEPS, PCHUNK, G)
    xspec = lambda s: pl.BlockSpec((PCHUNK, CS, B), lambda j, s=s: (j, s, 0))
    res = pl.pallas_call(
        _body,
        grid=(NSTEPS,),
        in_specs=[xspec(0), xspec(1), xspec(2), xspec(3),
                  pl.BlockSpec((1, PCHUNK, G), lambda j: (j, 0, 0))],
        out_specs=pl.BlockSpec((1, PCHUNK, G), lambda j: (j, 0, 0)),
        out_shape=jax.ShapeDtypeStruct((NSTEPS, PCHUNK, G), jnp.float32),
    )(x, x, x, x, gm_t)
    return res.reshape(P, G).T.reshape(G, 1, 7, 7)
